# Initial kernel scaffold; baseline (speedup 1.0000x reference)
import functools

import jax
import jax.numpy as jnp
from jax.experimental import pallas as pl
from jax.experimental.pallas import tpu as pltpu


def _sigmoid(x):
    return 1.0 / (1.0 + jnp.exp(-x))


def _softplus(x):
    return jnp.maximum(x, 0.0) + jnp.log(1.0 + jnp.exp(-jnp.abs(x)))


def _proj_kernel(x_ref, w_ref, b_ref, out_ref):
    out_ref[...] = (
        jnp.dot(x_ref[...], w_ref[...], preferred_element_type=jnp.float32)
        + b_ref[...])


def _start_idx_dma(ei_ref, idx_sm, sems, slot, start, be):
    pltpu.make_async_copy(
        ei_ref.at[:, pl.ds(start, be)], idx_sm.at[slot], sems.at[slot]).start()


def _wait_idx_dma(ei_ref, idx_sm, sems, slot, start, be):
    pltpu.make_async_copy(
        ei_ref.at[:, pl.ds(start, be)], idx_sm.at[slot], sems.at[slot]).wait()


def _msg_kernel(ei_ref, ea_ref, d_ref, p_ref, w5_ref, out_ref,
                z_scr, msg_scr, idx_sm, sems, *, be, nb, f):
    c = pl.program_id(0)
    j = pl.program_id(1)
    blk = c * nb + j
    slot = jax.lax.rem(j, 2)
    nxt = 1 - slot

    @pl.when(j == 0)
    def _():
        _start_idx_dma(ei_ref, idx_sm, sems, 0, blk * be, be)
        out_ref[...] = jnp.zeros_like(out_ref)

    @pl.when(j + 1 < nb)
    def _():
        _start_idx_dma(ei_ref, idx_sm, sems, nxt, (blk + 1) * be, be)

    _wait_idx_dma(ei_ref, idx_sm, sems, slot, blk * be, be)

    iota_z = jax.lax.broadcasted_iota(jnp.int32, (8, 2 * f), 0)
    iota_m = jax.lax.broadcasted_iota(jnp.int32, (8, f), 0)

    def gbody(k, _):
        k8 = pl.multiple_of(k * 8, 8)
        acc = jnp.zeros((8, 2 * f), jnp.float32)
        for u in range(8):
            si = idx_sm[slot, 0, k8 + u]
            di = idx_sm[slot, 1, k8 + u]
            sb = pl.multiple_of((si >> 3) << 3, 8)
            db = pl.multiple_of((di >> 3) << 3, 8)
            chd = p_ref[pl.ds(db, 8), 0:2 * f]
            chs = p_ref[pl.ds(sb, 8), 2 * f:4 * f]
            row = (pltpu.roll(chd, u - (di & 7), axis=0)
                   + pltpu.roll(chs, u - (si & 7), axis=0))
            acc = jnp.where(iota_z == u, row, acc)
        z_scr[pl.ds(k8, 8), :] = acc
        return 0

    jax.lax.fori_loop(0, be // 8, gbody, 0)

    z = z_scr[...] + jnp.dot(ea_ref[...], w5_ref[...],
                             preferred_element_type=jnp.float32)
    zf = z[:, 0:f]
    zs = z[:, f:2 * f]
    d = d_ref[...]
    g = jnp.exp(d * d * (-1.0 / 18.0))
    msg_scr[...] = _sigmoid(zf) * _softplus(zs) * g

    def sbody(k, _):
        k8 = pl.multiple_of(k * 8, 8)
        chm = msg_scr[pl.ds(k8, 8), :]
        for u in range(8):
            di = idx_sm[slot, 1, k8 + u]
            db = pl.multiple_of((di >> 3) << 3, 8)
            r = di & 7
            rolled = pltpu.roll(chm, r - u, axis=0)
            cur = out_ref[pl.ds(db, 8), :]
            out_ref[pl.ds(db, 8), :] = cur + jnp.where(iota_m == r, rolled, 0.0)
        return 0

    jax.lax.fori_loop(0, be // 8, sbody, 0)


def _fin_kernel(pa_ref, pb_ref, x_ref, w6_ref, atom_ref, ap1_ref, ap2_ref):
    a = pa_ref[...] + pb_ref[...] + x_ref[...]
    atom_ref[...] = a
    ap = jnp.dot(a, w6_ref[...], preferred_element_type=jnp.float32)
    ap1_ref[...] = ap[:, 0:16]
    ap2_ref[...] = ap[:, 16:32]


def _emlp_kernel(ei_ref, ea_ref, ap1_ref, ap2_ref, w7_ref, b1_ref,
                 w2_ref, b2_ref, out_ref, pre_scr, idx_sm, sems, *, be, nb):
    c = pl.program_id(0)
    j = pl.program_id(1)
    blk = c * nb + j
    slot = jax.lax.rem(j, 2)
    nxt = 1 - slot

    @pl.when(j == 0)
    def _():
        _start_idx_dma(ei_ref, idx_sm, sems, 0, blk * be, be)

    @pl.when(j + 1 < nb)
    def _():
        _start_idx_dma(ei_ref, idx_sm, sems, nxt, (blk + 1) * be, be)

    _wait_idx_dma(ei_ref, idx_sm, sems, slot, blk * be, be)

    iota_h = jax.lax.broadcasted_iota(jnp.int32, (8, 16), 0)

    def gbody(k, _):
        k8 = pl.multiple_of(k * 8, 8)
        acc = jnp.zeros((8, 16), jnp.float32)
        for u in range(8):
            si = idx_sm[slot, 0, k8 + u]
            di = idx_sm[slot, 1, k8 + u]
            sb = pl.multiple_of((si >> 3) << 3, 8)
            db = pl.multiple_of((di >> 3) << 3, 8)
            ch1 = ap1_ref[pl.ds(sb, 8), :]
            ch2 = ap2_ref[pl.ds(db, 8), :]
            row = (pltpu.roll(ch1, u - (si & 7), axis=0)
                   + pltpu.roll(ch2, u - (di & 7), axis=0))
            acc = jnp.where(iota_h == u, row, acc)
        pre_scr[pl.ds(k8, 8), :] = acc
        return 0

    jax.lax.fori_loop(0, be // 8, gbody, 0)

    pre = (pre_scr[...]
           + jnp.dot(ea_ref[...], w7_ref[...],
                     preferred_element_type=jnp.float32)
           + b1_ref[...])
    h = pre * _sigmoid(pre)
    o = jnp.dot(h, w2_ref[...], preferred_element_type=jnp.float32) + b2_ref[...]
    out_ref[...] = o * _sigmoid(o)


def kernel(atom_fea, edge_index, edge_fea, distance,
           wf, bf, ws, bs, w1, b1, w2, b2):
    N, F = atom_fea.shape
    E, D = edge_fea.shape
    H = w1.shape[1]
    Dout = w2.shape[1]
    Hp = 16

    x = atom_fea.astype(jnp.float32)
    ea = edge_fea.astype(jnp.float32)
    dd = distance.astype(jnp.float32).reshape(E, 1)
    ei = edge_index.astype(jnp.int32)

    wf = wf.astype(jnp.float32)
    ws = ws.astype(jnp.float32)
    w1 = w1.astype(jnp.float32)
    w2 = w2.astype(jnp.float32)

    w4 = jnp.concatenate([wf[0:F], ws[0:F], wf[F:2 * F], ws[F:2 * F]], axis=1)
    b4 = jnp.concatenate([bf.astype(jnp.float32), bs.astype(jnp.float32),
                          jnp.zeros((2 * F,), jnp.float32)]).reshape(1, 4 * F)
    w5 = jnp.concatenate([wf[2 * F:], ws[2 * F:]], axis=1)

    w11p = jnp.pad(w1[0:F], ((0, 0), (0, Hp - H)))
    w12p = jnp.pad(w1[F:2 * F], ((0, 0), (0, Hp - H)))
    w6 = jnp.concatenate([w11p, w12p], axis=1)
    w7 = jnp.pad(w1[2 * F:], ((0, 0), (0, Hp - H)))
    b1p = jnp.pad(b1.astype(jnp.float32), (0, Hp - H)).reshape(1, Hp)
    w2p = jnp.pad(w2, ((0, Hp - H), (0, 0)))
    b2r = b2.astype(jnp.float32).reshape(1, Dout)

    bn = N // 2
    p_nodes = pl.pallas_call(
        _proj_kernel,
        out_shape=jax.ShapeDtypeStruct((N, 4 * F), jnp.float32),
        grid=(2,),
        in_specs=[
            pl.BlockSpec((bn, F), lambda i: (i, 0)),
            pl.BlockSpec((F, 4 * F), lambda i: (0, 0)),
            pl.BlockSpec((1, 4 * F), lambda i: (0, 0)),
        ],
        out_specs=pl.BlockSpec((bn, 4 * F), lambda i: (i, 0)),
        compiler_params=pltpu.CompilerParams(
            dimension_semantics=("parallel",)),
    )(x, w4, b4)

    be = 1024 if E % 2048 == 0 else E // 2
    nb = E // (2 * be)
    part = pl.pallas_call(
        functools.partial(_msg_kernel, be=be, nb=nb, f=F),
        out_shape=jax.ShapeDtypeStruct((2 * N, F), jnp.float32),
        grid=(2, nb),
        in_specs=[
            pl.BlockSpec(memory_space=pl.ANY),
            pl.BlockSpec((be, D), lambda c, j: (c * nb + j, 0)),
            pl.BlockSpec((be, 1), lambda c, j: (c * nb + j, 0)),
            pl.BlockSpec((N, 4 * F), lambda c, j: (0, 0)),
            pl.BlockSpec((D, 2 * F), lambda c, j: (0, 0)),
        ],
        out_specs=pl.BlockSpec((N, F), lambda c, j: (c, 0)),
        scratch_shapes=[
            pltpu.VMEM((be, 2 * F), jnp.float32),
            pltpu.VMEM((be, F), jnp.float32),
            pltpu.SMEM((2, 2, be), jnp.int32),
            pltpu.SemaphoreType.DMA((2,)),
        ],
        compiler_params=pltpu.CompilerParams(
            dimension_semantics=("parallel", "arbitrary")),
    )(ei, ea, dd, p_nodes, w5)

    bn0 = N // 8
    atom_out, ap1, ap2 = pl.pallas_call(
        _fin_kernel,
        out_shape=(
            jax.ShapeDtypeStruct((N, F), jnp.float32),
            jax.ShapeDtypeStruct((N, Hp), jnp.float32),
            jax.ShapeDtypeStruct((N, Hp), jnp.float32),
        ),
        grid=(8,),
        in_specs=[
            pl.BlockSpec((bn0, F), lambda i: (i, 0)),
            pl.BlockSpec((bn0, F), lambda i: (8 + i, 0)),
            pl.BlockSpec((bn0, F), lambda i: (i, 0)),
            pl.BlockSpec((F, 2 * Hp), lambda i: (0, 0)),
        ],
        out_specs=(
            pl.BlockSpec((bn0, F), lambda i: (i, 0)),
            pl.BlockSpec((bn0, Hp), lambda i: (i, 0)),
            pl.BlockSpec((bn0, Hp), lambda i: (i, 0)),
        ),
        compiler_params=pltpu.CompilerParams(
            dimension_semantics=("parallel",)),
    )(part, part, x, w6)

    edge_out = pl.pallas_call(
        functools.partial(_emlp_kernel, be=be, nb=nb),
        out_shape=jax.ShapeDtypeStruct((E, Dout), jnp.float32),
        grid=(2, nb),
        in_specs=[
            pl.BlockSpec(memory_space=pl.ANY),
            pl.BlockSpec((be, D), lambda c, j: (c * nb + j, 0)),
            pl.BlockSpec((N, Hp), lambda c, j: (0, 0)),
            pl.BlockSpec((N, Hp), lambda c, j: (0, 0)),
            pl.BlockSpec((D, Hp), lambda c, j: (0, 0)),
            pl.BlockSpec((1, Hp), lambda c, j: (0, 0)),
            pl.BlockSpec((Hp, Dout), lambda c, j: (0, 0)),
            pl.BlockSpec((1, Dout), lambda c, j: (0, 0)),
        ],
        out_specs=pl.BlockSpec((be, Dout), lambda c, j: (c * nb + j, 0)),
        scratch_shapes=[
            pltpu.VMEM((be, Hp), jnp.float32),
            pltpu.SMEM((2, 2, be), jnp.int32),
            pltpu.SemaphoreType.DMA((2,)),
        ],
        compiler_params=pltpu.CompilerParams(
            dimension_semantics=("parallel", "arbitrary")),
    )(ei, ea, ap1, ap2, w7, b1p, w2p, b2r)

    return atom_out, edge_out


# speedup vs baseline: 3.6893x; 3.6893x over previous
import functools

import jax
import jax.numpy as jnp
from jax.experimental import pallas as pl
from jax.experimental.pallas import tpu as pltpu


def _sigmoid(x):
    return 1.0 / (1.0 + jnp.exp(-x))


def _softplus(x):
    return jnp.maximum(x, 0.0) + jnp.log(1.0 + jnp.exp(-jnp.abs(x)))


def _proj_kernel(x_ref, w_ref, b_ref, out_ref):
    out_ref[...] = (
        jnp.dot(x_ref[...], w_ref[...], preferred_element_type=jnp.float32)
        + b_ref[...])


def _start_idx_dma(ei_ref, idx_sm, sems, slot, start, be):
    pltpu.make_async_copy(
        ei_ref.at[pl.ds(pl.multiple_of(start, 1024), be)],
        idx_sm.at[pl.ds(slot * be, be)], sems.at[slot]).start()


def _wait_idx_dma(ei_ref, idx_sm, sems, slot, start, be):
    pltpu.make_async_copy(
        ei_ref.at[pl.ds(pl.multiple_of(start, 1024), be)],
        idx_sm.at[pl.ds(slot * be, be)], sems.at[slot]).wait()


def _tree_sum(vals):
    while len(vals) > 1:
        nxt = [a + b for a, b in zip(vals[0::2], vals[1::2])]
        if len(vals) % 2:
            nxt.append(vals[-1])
        vals = nxt
    return vals[0]


def _msg_kernel(ei_ref, ea_ref, d_ref, p_ref, w5_ref,
                outa_ref, outb_ref, outc_ref, outd_ref,
                z_scr, msg_scr, idx_sm, sems, *, be, nb, f):
    c = pl.program_id(0)
    j = pl.program_id(1)
    blk = c * nb + j
    slot = jax.lax.rem(j, 2)
    nxt = 1 - slot

    @pl.when(j == 0)
    def _():
        _start_idx_dma(ei_ref, idx_sm, sems, 0, blk * be, be)
        outa_ref[...] = jnp.zeros_like(outa_ref)
        outb_ref[...] = jnp.zeros_like(outb_ref)
        outc_ref[...] = jnp.zeros_like(outc_ref)
        outd_ref[...] = jnp.zeros_like(outd_ref)

    @pl.when(j + 1 < nb)
    def _():
        _start_idx_dma(ei_ref, idx_sm, sems, nxt, (blk + 1) * be, be)

    _wait_idx_dma(ei_ref, idx_sm, sems, slot, blk * be, be)

    iota_z = jax.lax.broadcasted_iota(jnp.int32, (8, 2 * f), 0)
    masks = [iota_z == u for u in range(8)]
    sbase = slot * be

    def gbody(k, _):
        k32 = pl.multiple_of(k * 32, 8)
        for cc in range(4):
            k8 = pl.multiple_of(k32 + cc * 8, 8)
            base = sbase + k8
            rows = []
            for u in range(8):
                pk = idx_sm[base + u]
                si = pk & 16383
                di = pk >> 14
                row = p_ref[di, 0, 0:2 * f] + p_ref[si, 0, 2 * f:4 * f]
                rows.append(jnp.where(masks[u], row, 0.0))
            z_scr[pl.ds(k8, 8), :] = _tree_sum(rows)
        return 0

    jax.lax.fori_loop(0, be // 32, gbody, 0)

    q = jnp.dot(ea_ref[...], w5_ref[...], preferred_element_type=jnp.float32)
    z = z_scr[...] + q
    zf = z[:, 0:f]
    zs = z[:, f:2 * f]
    d = d_ref[...]
    g = jnp.exp(d * d * (-1.0 / 18.0))
    msg_scr[...] = _sigmoid(zf) * _softplus(zs) * g

    bufs = [outa_ref, outb_ref, outc_ref, outd_ref]

    def sbody(k, _):
        k16 = pl.multiple_of(k * 16, 8)
        for cc in range(2):
            k8 = pl.multiple_of(k16 + cc * 8, 8)
            base = sbase + k8
            chm = msg_scr[pl.ds(k8, 8), :]
            for u in range(8):
                di = idx_sm[base + u] >> 14
                row = chm[u]
                tgt = bufs[u % 4]
                tgt[di, 0, 0:f] = tgt[di, 0, 0:f] + row
        return 0

    jax.lax.fori_loop(0, be // 16, sbody, 0)


def _fin_kernel(pa0_ref, pa1_ref, pb0_ref, pb1_ref,
                pc0_ref, pc1_ref, pd0_ref, pd1_ref, x_ref, w6_ref,
                atom_ref, ap1_ref, ap2_ref):
    fdim = x_ref.shape[1]
    psum = (((pa0_ref[...] + pa1_ref[...]) + (pb0_ref[...] + pb1_ref[...]))
            + ((pc0_ref[...] + pc1_ref[...]) + (pd0_ref[...] + pd1_ref[...])))
    a = psum[:, 0:fdim] + x_ref[...]
    atom_ref[...] = a
    ap = jnp.dot(a, w6_ref[...], preferred_element_type=jnp.float32)
    ap1_ref[...] = ap[:, 0:16]
    ap2_ref[...] = ap[:, 16:32]


def _emlp_kernel(ei_ref, ea_ref, ap1_ref, ap2_ref, w7_ref, b1_ref,
                 w2_ref, b2_ref, out_ref, pre_scr, idx_sm, sems, *, be, nb):
    c = pl.program_id(0)
    j = pl.program_id(1)
    blk = c * nb + j
    slot = jax.lax.rem(j, 2)
    nxt = 1 - slot

    @pl.when(j == 0)
    def _():
        _start_idx_dma(ei_ref, idx_sm, sems, 0, blk * be, be)

    @pl.when(j + 1 < nb)
    def _():
        _start_idx_dma(ei_ref, idx_sm, sems, nxt, (blk + 1) * be, be)

    _wait_idx_dma(ei_ref, idx_sm, sems, slot, blk * be, be)

    iota_h = jax.lax.broadcasted_iota(jnp.int32, (8, 16), 0)
    masks = [iota_h == u for u in range(8)]
    sbase = slot * be

    def gbody(k, _):
        k32 = pl.multiple_of(k * 32, 8)
        for cc in range(4):
            k8 = pl.multiple_of(k32 + cc * 8, 8)
            base = sbase + k8
            rows = []
            for u in range(8):
                pk = idx_sm[base + u]
                si = pk & 16383
                di = pk >> 14
                row = ap1_ref[si, 0, :] + ap2_ref[di, 0, :]
                rows.append(jnp.where(masks[u], row, 0.0))
            pre_scr[pl.ds(k8, 8), :] = _tree_sum(rows)
        return 0

    jax.lax.fori_loop(0, be // 32, gbody, 0)

    pre = (pre_scr[...]
           + jnp.dot(ea_ref[...], w7_ref[...],
                     preferred_element_type=jnp.float32)
           + b1_ref[...])
    h = pre * _sigmoid(pre)
    o = jnp.dot(h, w2_ref[...], preferred_element_type=jnp.float32) + b2_ref[...]
    out_ref[...] = o * _sigmoid(o)


def kernel(atom_fea, edge_index, edge_fea, distance,
           wf, bf, ws, bs, w1, b1, w2, b2):
    N, F = atom_fea.shape
    E, D = edge_fea.shape
    H = w1.shape[1]
    Dout = w2.shape[1]
    Hp = 16

    x = atom_fea.astype(jnp.float32)
    ea = edge_fea.astype(jnp.float32)
    dd = distance.astype(jnp.float32).reshape(E, 1)
    eidx = edge_index.astype(jnp.int32)
    ei = (eidx[1] << 14) | eidx[0]

    wf = wf.astype(jnp.float32)
    ws = ws.astype(jnp.float32)
    w1 = w1.astype(jnp.float32)
    w2 = w2.astype(jnp.float32)

    w4 = jnp.concatenate([wf[0:F], ws[0:F], wf[F:2 * F], ws[F:2 * F]], axis=1)
    b4 = jnp.concatenate([bf.astype(jnp.float32), bs.astype(jnp.float32),
                          jnp.zeros((2 * F,), jnp.float32)]).reshape(1, 4 * F)
    w5 = jnp.concatenate([wf[2 * F:], ws[2 * F:]], axis=1)

    w11p = jnp.pad(w1[0:F], ((0, 0), (0, Hp - H)))
    w12p = jnp.pad(w1[F:2 * F], ((0, 0), (0, Hp - H)))
    w6 = jnp.concatenate([w11p, w12p], axis=1)
    w7 = jnp.pad(w1[2 * F:], ((0, 0), (0, Hp - H)))
    b1p = jnp.pad(b1.astype(jnp.float32), (0, Hp - H)).reshape(1, Hp)
    w2p = jnp.pad(w2, ((0, Hp - H), (0, 0)))
    b2r = b2.astype(jnp.float32).reshape(1, Dout)

    bn = N // 2
    p_nodes = pl.pallas_call(
        _proj_kernel,
        out_shape=jax.ShapeDtypeStruct((N, 4 * F), jnp.float32),
        grid=(2,),
        in_specs=[
            pl.BlockSpec((bn, F), lambda i: (i, 0)),
            pl.BlockSpec((F, 4 * F), lambda i: (0, 0)),
            pl.BlockSpec((1, 4 * F), lambda i: (0, 0)),
        ],
        out_specs=pl.BlockSpec((bn, 4 * F), lambda i: (i, 0)),
        compiler_params=pltpu.CompilerParams(
            dimension_semantics=("parallel",)),
    )(x, w4, b4)
    p_nodes = p_nodes.reshape(N, 1, 4 * F)

    be = 1024 if E % 2048 == 0 else E // 2
    nb = E // (2 * be)
    parts = pl.pallas_call(
        functools.partial(_msg_kernel, be=be, nb=nb, f=F),
        out_shape=tuple(
            jax.ShapeDtypeStruct((2 * N, 1, 2 * F), jnp.float32)
            for _ in range(4)),
        grid=(2, nb),
        in_specs=[
            pl.BlockSpec(memory_space=pl.ANY),
            pl.BlockSpec((be, D), lambda c, j: (c * nb + j, 0)),
            pl.BlockSpec((be, 1), lambda c, j: (c * nb + j, 0)),
            pl.BlockSpec((N, 1, 4 * F), lambda c, j: (0, 0, 0)),
            pl.BlockSpec((D, 2 * F), lambda c, j: (0, 0)),
        ],
        out_specs=tuple(
            pl.BlockSpec((N, 1, 2 * F), lambda c, j: (c, 0, 0))
            for _ in range(4)),
        scratch_shapes=[
            pltpu.VMEM((be, 2 * F), jnp.float32),
            pltpu.VMEM((be, F), jnp.float32),
            pltpu.SMEM((2 * be,), jnp.int32),
            pltpu.SemaphoreType.DMA((2,)),
        ],
        compiler_params=pltpu.CompilerParams(
            dimension_semantics=("parallel", "arbitrary")),
    )(ei, ea, dd, p_nodes, w5)
    parts2 = [p.reshape(2 * N, 2 * F) for p in parts]

    bn0 = N // 8
    atom_out, ap1, ap2 = pl.pallas_call(
        _fin_kernel,
        out_shape=(
            jax.ShapeDtypeStruct((N, F), jnp.float32),
            jax.ShapeDtypeStruct((N, Hp), jnp.float32),
            jax.ShapeDtypeStruct((N, Hp), jnp.float32),
        ),
        grid=(8,),
        in_specs=(
            [pl.BlockSpec((bn0, 2 * F), lambda i: (i, 0)),
             pl.BlockSpec((bn0, 2 * F), lambda i: (8 + i, 0))] * 4
            + [pl.BlockSpec((bn0, F), lambda i: (i, 0)),
               pl.BlockSpec((F, 2 * Hp), lambda i: (0, 0))]),
        out_specs=(
            pl.BlockSpec((bn0, F), lambda i: (i, 0)),
            pl.BlockSpec((bn0, Hp), lambda i: (i, 0)),
            pl.BlockSpec((bn0, Hp), lambda i: (i, 0)),
        ),
        compiler_params=pltpu.CompilerParams(
            dimension_semantics=("parallel",)),
    )(parts2[0], parts2[0], parts2[1], parts2[1],
      parts2[2], parts2[2], parts2[3], parts2[3], x, w6)
    ap1 = ap1.reshape(N, 1, Hp)
    ap2 = ap2.reshape(N, 1, Hp)

    edge_out = pl.pallas_call(
        functools.partial(_emlp_kernel, be=be, nb=nb),
        out_shape=jax.ShapeDtypeStruct((E, Dout), jnp.float32),
        grid=(2, nb),
        in_specs=[
            pl.BlockSpec(memory_space=pl.ANY),
            pl.BlockSpec((be, D), lambda c, j: (c * nb + j, 0)),
            pl.BlockSpec((N, 1, Hp), lambda c, j: (0, 0, 0)),
            pl.BlockSpec((N, 1, Hp), lambda c, j: (0, 0, 0)),
            pl.BlockSpec((D, Hp), lambda c, j: (0, 0)),
            pl.BlockSpec((1, Hp), lambda c, j: (0, 0)),
            pl.BlockSpec((Hp, Dout), lambda c, j: (0, 0)),
            pl.BlockSpec((1, Dout), lambda c, j: (0, 0)),
        ],
        out_specs=pl.BlockSpec((be, Dout), lambda c, j: (c * nb + j, 0)),
        scratch_shapes=[
            pltpu.VMEM((be, Hp), jnp.float32),
            pltpu.SMEM((2 * be,), jnp.int32),
            pltpu.SemaphoreType.DMA((2,)),
        ],
        compiler_params=pltpu.CompilerParams(
            dimension_semantics=("parallel", "arbitrary")),
    )(ei, ea, ap1, ap2, w7, b1p, w2p, b2r)

    return atom_out, edge_out


# speedup vs baseline: 3.8347x; 1.0394x over previous
import functools

import jax
import jax.numpy as jnp
from jax.experimental import pallas as pl
from jax.experimental.pallas import tpu as pltpu


def _sigmoid(x):
    return 1.0 / (1.0 + jnp.exp(-x))


def _softplus(x):
    return jnp.maximum(x, 0.0) + jnp.log(1.0 + jnp.exp(-jnp.abs(x)))


def _proj_kernel(x_ref, w_ref, b_ref, out_ref):
    out_ref[...] = (
        jnp.dot(x_ref[...], w_ref[...], preferred_element_type=jnp.float32)
        + b_ref[...])


def _start_idx_dma(ei_ref, idx_sm, sems, slot, start, be):
    pltpu.make_async_copy(
        ei_ref.at[pl.ds(pl.multiple_of(start, 1024), be)],
        idx_sm.at[pl.ds(slot * be, be)], sems.at[slot]).start()


def _wait_idx_dma(ei_ref, idx_sm, sems, slot, start, be):
    pltpu.make_async_copy(
        ei_ref.at[pl.ds(pl.multiple_of(start, 1024), be)],
        idx_sm.at[pl.ds(slot * be, be)], sems.at[slot]).wait()


def _tree_sum(vals):
    while len(vals) > 1:
        nxt = [a + b for a, b in zip(vals[0::2], vals[1::2])]
        if len(vals) % 2:
            nxt.append(vals[-1])
        vals = nxt
    return vals[0]


def _msg_kernel(ei_ref, ea_ref, d_ref, p_ref, w5_ref,
                outa_ref, outb_ref, outc_ref, outd_ref,
                z_scr, msg_scr, idx_sm, sems, *, be, nb, f):
    c = pl.program_id(0)
    j = pl.program_id(1)
    blk = c * nb + j
    slot = jax.lax.rem(j, 3)
    nxt = jax.lax.rem(j + 1, 3)
    prv = jax.lax.rem(j + 2, 3)

    @pl.when(j == 0)
    def _():
        _start_idx_dma(ei_ref, idx_sm, sems, 0, blk * be, be)
        outa_ref[...] = jnp.zeros_like(outa_ref)
        outb_ref[...] = jnp.zeros_like(outb_ref)
        outc_ref[...] = jnp.zeros_like(outc_ref)
        outd_ref[...] = jnp.zeros_like(outd_ref)

    @pl.when(j + 1 < nb)
    def _():
        _start_idx_dma(ei_ref, idx_sm, sems, nxt, (blk + 1) * be, be)

    @pl.when(j < nb)
    def _():
        _wait_idx_dma(ei_ref, idx_sm, sems, slot, blk * be, be)

    iota_z = jax.lax.broadcasted_iota(jnp.int32, (8, 2 * f), 0)
    masks = [iota_z == u for u in range(8)]
    sbase = slot * be
    pbase = prv * be
    mslot = jax.lax.rem(j, 2)
    mprv = 1 - mslot
    bufs = [outa_ref, outb_ref, outc_ref, outd_ref]

    def gather_chunk(k8, base):
        rows = []
        for u in range(8):
            pk = idx_sm[base + u]
            si = pk & 16383
            di = pk >> 14
            row = p_ref[di, 0, 0:2 * f] + p_ref[si, 0, 2 * f:4 * f]
            rows.append(jnp.where(masks[u], row, 0.0))
        z_scr[pl.ds(k8, 8), :] = _tree_sum(rows)

    def scatter_chunk(k8, base):
        chm = msg_scr[mprv, pl.ds(k8, 8), :]
        for u in range(8):
            di = idx_sm[base + u] >> 14
            row = chm[u]
            tgt = bufs[u % 4]
            tgt[di, 0, 0:f] = tgt[di, 0, 0:f] + row

    @pl.when((j > 0) & (j < nb))
    def _():
        def fbody(k, _):
            k16 = pl.multiple_of(k * 16, 8)
            for cc in range(2):
                k8 = pl.multiple_of(k16 + cc * 8, 8)
                gather_chunk(k8, sbase + k8)
                scatter_chunk(k8, pbase + k8)
            return 0
        jax.lax.fori_loop(0, be // 16, fbody, 0)

    @pl.when(j == 0)
    def _():
        def gb(k, _):
            k16 = pl.multiple_of(k * 16, 8)
            for cc in range(2):
                k8 = pl.multiple_of(k16 + cc * 8, 8)
                gather_chunk(k8, sbase + k8)
            return 0
        jax.lax.fori_loop(0, be // 16, gb, 0)

    @pl.when(j == nb)
    def _():
        def sb(k, _):
            k16 = pl.multiple_of(k * 16, 8)
            for cc in range(2):
                k8 = pl.multiple_of(k16 + cc * 8, 8)
                scatter_chunk(k8, pbase + k8)
            return 0
        jax.lax.fori_loop(0, be // 16, sb, 0)

    @pl.when(j < nb)
    def _():
        q = jnp.dot(ea_ref[...], w5_ref[...],
                    preferred_element_type=jnp.float32)
        z = z_scr[...] + q
        zf = z[:, 0:f]
        zs = z[:, f:2 * f]
        d = d_ref[...]
        g = jnp.exp(d * d * (-1.0 / 18.0))
        msg_scr[mslot, :, :] = _sigmoid(zf) * _softplus(zs) * g


def _fin_kernel(pa0_ref, pa1_ref, pb0_ref, pb1_ref,
                pc0_ref, pc1_ref, pd0_ref, pd1_ref, x_ref, w6_ref,
                atom_ref, ap1_ref, ap2_ref):
    fdim = x_ref.shape[1]
    psum = (((pa0_ref[...] + pa1_ref[...]) + (pb0_ref[...] + pb1_ref[...]))
            + ((pc0_ref[...] + pc1_ref[...]) + (pd0_ref[...] + pd1_ref[...])))
    a = psum[:, 0:fdim] + x_ref[...]
    atom_ref[...] = a
    ap = jnp.dot(a, w6_ref[...], preferred_element_type=jnp.float32)
    ap1_ref[...] = ap[:, 0:16]
    ap2_ref[...] = ap[:, 16:32]


def _emlp_kernel(ei_ref, ea_ref, ap1_ref, ap2_ref, w7_ref, b1_ref,
                 w2_ref, b2_ref, out_ref, pre_scr, idx_sm, sems, *, be, nb):
    c = pl.program_id(0)
    j = pl.program_id(1)
    blk = c * nb + j
    slot = jax.lax.rem(j, 2)
    nxt = 1 - slot

    @pl.when(j == 0)
    def _():
        _start_idx_dma(ei_ref, idx_sm, sems, 0, blk * be, be)

    @pl.when(j + 1 < nb)
    def _():
        _start_idx_dma(ei_ref, idx_sm, sems, nxt, (blk + 1) * be, be)

    _wait_idx_dma(ei_ref, idx_sm, sems, slot, blk * be, be)

    iota_h = jax.lax.broadcasted_iota(jnp.int32, (8, 16), 0)
    masks = [iota_h == u for u in range(8)]
    sbase = slot * be

    def gbody(k, _):
        k32 = pl.multiple_of(k * 32, 8)
        for cc in range(4):
            k8 = pl.multiple_of(k32 + cc * 8, 8)
            base = sbase + k8
            rows = []
            for u in range(8):
                pk = idx_sm[base + u]
                si = pk & 16383
                di = pk >> 14
                row = ap1_ref[si, 0, :] + ap2_ref[di, 0, :]
                rows.append(jnp.where(masks[u], row, 0.0))
            pre_scr[pl.ds(k8, 8), :] = _tree_sum(rows)
        return 0

    jax.lax.fori_loop(0, be // 32, gbody, 0)

    pre = (pre_scr[...]
           + jnp.dot(ea_ref[...], w7_ref[...],
                     preferred_element_type=jnp.float32)
           + b1_ref[...])
    h = pre * _sigmoid(pre)
    o = jnp.dot(h, w2_ref[...], preferred_element_type=jnp.float32) + b2_ref[...]
    out_ref[...] = o * _sigmoid(o)


def kernel(atom_fea, edge_index, edge_fea, distance,
           wf, bf, ws, bs, w1, b1, w2, b2):
    N, F = atom_fea.shape
    E, D = edge_fea.shape
    H = w1.shape[1]
    Dout = w2.shape[1]
    Hp = 16

    x = atom_fea.astype(jnp.float32)
    ea = edge_fea.astype(jnp.float32)
    dd = distance.astype(jnp.float32).reshape(E, 1)
    eidx = edge_index.astype(jnp.int32)
    ei = (eidx[1] << 14) | eidx[0]

    wf = wf.astype(jnp.float32)
    ws = ws.astype(jnp.float32)
    w1 = w1.astype(jnp.float32)
    w2 = w2.astype(jnp.float32)

    w4 = jnp.concatenate([wf[0:F], ws[0:F], wf[F:2 * F], ws[F:2 * F]], axis=1)
    b4 = jnp.concatenate([bf.astype(jnp.float32), bs.astype(jnp.float32),
                          jnp.zeros((2 * F,), jnp.float32)]).reshape(1, 4 * F)
    w5 = jnp.concatenate([wf[2 * F:], ws[2 * F:]], axis=1)

    w11p = jnp.pad(w1[0:F], ((0, 0), (0, Hp - H)))
    w12p = jnp.pad(w1[F:2 * F], ((0, 0), (0, Hp - H)))
    w6 = jnp.concatenate([w11p, w12p], axis=1)
    w7 = jnp.pad(w1[2 * F:], ((0, 0), (0, Hp - H)))
    b1p = jnp.pad(b1.astype(jnp.float32), (0, Hp - H)).reshape(1, Hp)
    w2p = jnp.pad(w2, ((0, Hp - H), (0, 0)))
    b2r = b2.astype(jnp.float32).reshape(1, Dout)

    bn = N // 2
    p_nodes = pl.pallas_call(
        _proj_kernel,
        out_shape=jax.ShapeDtypeStruct((N, 4 * F), jnp.float32),
        grid=(2,),
        in_specs=[
            pl.BlockSpec((bn, F), lambda i: (i, 0)),
            pl.BlockSpec((F, 4 * F), lambda i: (0, 0)),
            pl.BlockSpec((1, 4 * F), lambda i: (0, 0)),
        ],
        out_specs=pl.BlockSpec((bn, 4 * F), lambda i: (i, 0)),
        compiler_params=pltpu.CompilerParams(
            dimension_semantics=("arbitrary",)),
    )(x, w4, b4)
    p_nodes = p_nodes.reshape(N, 1, 4 * F)

    be = 1024 if E % 2048 == 0 else E // 2
    nb = E // (2 * be)
    parts = pl.pallas_call(
        functools.partial(_msg_kernel, be=be, nb=nb, f=F),
        out_shape=tuple(
            jax.ShapeDtypeStruct((2 * N, 1, 2 * F), jnp.float32)
            for _ in range(4)),
        grid=(2, nb + 1),
        in_specs=[
            pl.BlockSpec(memory_space=pl.ANY),
            pl.BlockSpec((be, D),
                         lambda c, j: (c * nb + jnp.minimum(j, nb - 1), 0)),
            pl.BlockSpec((be, 1),
                         lambda c, j: (c * nb + jnp.minimum(j, nb - 1), 0)),
            pl.BlockSpec((N, 1, 4 * F), lambda c, j: (0, 0, 0)),
            pl.BlockSpec((D, 2 * F), lambda c, j: (0, 0)),
        ],
        out_specs=tuple(
            pl.BlockSpec((N, 1, 2 * F), lambda c, j: (c, 0, 0))
            for _ in range(4)),
        scratch_shapes=[
            pltpu.VMEM((be, 2 * F), jnp.float32),
            pltpu.VMEM((2, be, F), jnp.float32),
            pltpu.SMEM((3 * be,), jnp.int32),
            pltpu.SemaphoreType.DMA((3,)),
        ],
        compiler_params=pltpu.CompilerParams(
            dimension_semantics=("arbitrary", "arbitrary")),
    )(ei, ea, dd, p_nodes, w5)
    parts2 = [p.reshape(2 * N, 2 * F) for p in parts]

    bn0 = N // 8
    atom_out, ap1, ap2 = pl.pallas_call(
        _fin_kernel,
        out_shape=(
            jax.ShapeDtypeStruct((N, F), jnp.float32),
            jax.ShapeDtypeStruct((N, Hp), jnp.float32),
            jax.ShapeDtypeStruct((N, Hp), jnp.float32),
        ),
        grid=(2, 4),
        in_specs=(
            [pl.BlockSpec((bn0, 2 * F), lambda c, i: (c * 4 + i, 0)),
             pl.BlockSpec((bn0, 2 * F), lambda c, i: (8 + c * 4 + i, 0))] * 4
            + [pl.BlockSpec((bn0, F), lambda c, i: (c * 4 + i, 0)),
               pl.BlockSpec((F, 2 * Hp), lambda c, i: (0, 0))]),
        out_specs=(
            pl.BlockSpec((bn0, F), lambda c, i: (c * 4 + i, 0)),
            pl.BlockSpec((bn0, Hp), lambda c, i: (c * 4 + i, 0)),
            pl.BlockSpec((bn0, Hp), lambda c, i: (c * 4 + i, 0)),
        ),
        compiler_params=pltpu.CompilerParams(
            dimension_semantics=("arbitrary", "arbitrary")),
    )(parts2[0], parts2[0], parts2[1], parts2[1],
      parts2[2], parts2[2], parts2[3], parts2[3], x, w6)
    ap1 = ap1.reshape(N, 1, Hp)
    ap2 = ap2.reshape(N, 1, Hp)

    edge_out = pl.pallas_call(
        functools.partial(_emlp_kernel, be=be, nb=nb),
        out_shape=jax.ShapeDtypeStruct((E, Dout), jnp.float32),
        grid=(2, nb),
        in_specs=[
            pl.BlockSpec(memory_space=pl.ANY),
            pl.BlockSpec((be, D), lambda c, j: (c * nb + j, 0)),
            pl.BlockSpec((N, 1, Hp), lambda c, j: (0, 0, 0)),
            pl.BlockSpec((N, 1, Hp), lambda c, j: (0, 0, 0)),
            pl.BlockSpec((D, Hp), lambda c, j: (0, 0)),
            pl.BlockSpec((1, Hp), lambda c, j: (0, 0)),
            pl.BlockSpec((Hp, Dout), lambda c, j: (0, 0)),
            pl.BlockSpec((1, Dout), lambda c, j: (0, 0)),
        ],
        out_specs=pl.BlockSpec((be, Dout), lambda c, j: (c * nb + j, 0)),
        scratch_shapes=[
            pltpu.VMEM((be, Hp), jnp.float32),
            pltpu.SMEM((2 * be,), jnp.int32),
            pltpu.SemaphoreType.DMA((2,)),
        ],
        compiler_params=pltpu.CompilerParams(
            dimension_semantics=("arbitrary", "arbitrary")),
    )(ei, ea, ap1, ap2, w7, b1p, w2p, b2r)

    return atom_out, edge_out


# speedup vs baseline: 4.0275x; 1.0503x over previous
import functools

import jax
import jax.numpy as jnp
from jax.experimental import pallas as pl
from jax.experimental.pallas import tpu as pltpu


def _sigmoid(x):
    return 1.0 / (1.0 + jnp.exp(-x))


def _softplus(x):
    return jnp.maximum(x, 0.0) + jnp.log(1.0 + jnp.exp(-jnp.abs(x)))


def _proj_kernel(x_ref, w_ref, b_ref, out_ref):
    out_ref[...] = (
        jnp.dot(x_ref[...], w_ref[...], preferred_element_type=jnp.float32)
        + b_ref[...])


def _start_idx_dma(ei_ref, idx_sm, sems, slot, start, be):
    pltpu.make_async_copy(
        ei_ref.at[pl.ds(pl.multiple_of(start, 1024), be)],
        idx_sm.at[pl.ds(slot * be, be)], sems.at[slot]).start()


def _wait_idx_dma(ei_ref, idx_sm, sems, slot, start, be):
    pltpu.make_async_copy(
        ei_ref.at[pl.ds(pl.multiple_of(start, 1024), be)],
        idx_sm.at[pl.ds(slot * be, be)], sems.at[slot]).wait()


def _tree_sum(vals):
    while len(vals) > 1:
        nxt = [a + b for a, b in zip(vals[0::2], vals[1::2])]
        if len(vals) % 2:
            nxt.append(vals[-1])
        vals = nxt
    return vals[0]


def _msg_kernel(ei_ref, ea_ref, d_ref, p_ref, w5_ref,
                outa_ref, outb_ref, outc_ref, outd_ref,
                z_scr, msg_scr, idx_sm, sems, *, be, nb, f):
    c = pl.program_id(0)
    j = pl.program_id(1)
    blk = c * nb + j
    slot = jax.lax.rem(j, 3)
    nxt = jax.lax.rem(j + 1, 3)
    prv = jax.lax.rem(j + 2, 3)

    @pl.when(j == 0)
    def _():
        _start_idx_dma(ei_ref, idx_sm, sems, 0, blk * be, be)
        outa_ref[...] = jnp.zeros_like(outa_ref)
        outb_ref[...] = jnp.zeros_like(outb_ref)
        outc_ref[...] = jnp.zeros_like(outc_ref)
        outd_ref[...] = jnp.zeros_like(outd_ref)

    @pl.when(j + 1 < nb)
    def _():
        _start_idx_dma(ei_ref, idx_sm, sems, nxt, (blk + 1) * be, be)

    @pl.when(j < nb)
    def _():
        _wait_idx_dma(ei_ref, idx_sm, sems, slot, blk * be, be)

    iota_z = jax.lax.broadcasted_iota(jnp.int32, (8, 2 * f), 0)
    masks = [iota_z == u for u in range(8)]
    sbase = slot * be
    pbase = prv * be
    mslot = jax.lax.rem(j, 2)
    mprv = 1 - mslot
    bufs = [outa_ref, outb_ref, outc_ref, outd_ref]

    def gather_chunk(k8, base):
        rows = []
        for u in range(8):
            pk = idx_sm[base + u]
            si = pk & 16383
            di = pk >> 14
            row = p_ref[di, 0, 0:2 * f] + p_ref[si, 0, 2 * f:4 * f]
            rows.append(jnp.where(masks[u], row, 0.0))
        z_scr[pl.ds(k8, 8), :] = _tree_sum(rows)

    def scatter_chunk(k8, base):
        chm = msg_scr[mprv, pl.ds(k8, 8), :]
        for u in range(8):
            di = idx_sm[base + u] >> 14
            row = chm[u]
            tgt = bufs[u % 4]
            tgt[di, 0, 0:f] = tgt[di, 0, 0:f] + row

    @pl.when((j > 0) & (j < nb))
    def _():
        def fbody(k, _):
            k32 = pl.multiple_of(k * 32, 8)
            for cc in range(4):
                k8 = pl.multiple_of(k32 + cc * 8, 8)
                gather_chunk(k8, sbase + k8)
                scatter_chunk(k8, pbase + k8)
            return 0
        jax.lax.fori_loop(0, be // 32, fbody, 0)

    @pl.when(j == 0)
    def _():
        def gb(k, _):
            k16 = pl.multiple_of(k * 16, 8)
            for cc in range(2):
                k8 = pl.multiple_of(k16 + cc * 8, 8)
                gather_chunk(k8, sbase + k8)
            return 0
        jax.lax.fori_loop(0, be // 16, gb, 0)

    @pl.when(j == nb)
    def _():
        def sb(k, _):
            k16 = pl.multiple_of(k * 16, 8)
            for cc in range(2):
                k8 = pl.multiple_of(k16 + cc * 8, 8)
                scatter_chunk(k8, pbase + k8)
            return 0
        jax.lax.fori_loop(0, be // 16, sb, 0)

    @pl.when(j < nb)
    def _():
        q = jnp.dot(ea_ref[...], w5_ref[...],
                    preferred_element_type=jnp.float32)
        z = z_scr[...] + q
        zf = z[:, 0:f]
        zs = z[:, f:2 * f]
        d = d_ref[...]
        g = jnp.exp(d * d * (-1.0 / 18.0))
        msg_scr[mslot, :, :] = _sigmoid(zf) * _softplus(zs) * g


def _fin_kernel(pa0_ref, pa1_ref, pb0_ref, pb1_ref,
                pc0_ref, pc1_ref, pd0_ref, pd1_ref, x_ref, w6_ref,
                atom_ref, ap1_ref, ap2_ref):
    fdim = x_ref.shape[1]
    psum = (((pa0_ref[...] + pa1_ref[...]) + (pb0_ref[...] + pb1_ref[...]))
            + ((pc0_ref[...] + pc1_ref[...]) + (pd0_ref[...] + pd1_ref[...])))
    a = psum[:, 0:fdim] + x_ref[...]
    atom_ref[...] = a
    ap = jnp.dot(a, w6_ref[...], preferred_element_type=jnp.float32)
    ap1_ref[...] = ap[:, 0:16]
    ap2_ref[...] = ap[:, 16:32]


def _emlp_kernel(ei_ref, ea_ref, ap1_ref, ap2_ref, w7_ref, b1_ref,
                 w2_ref, b2_ref, out_ref, pre_scr, idx_sm, sems, *, be, nb):
    c = pl.program_id(0)
    j = pl.program_id(1)
    blk = c * nb + j
    slot = jax.lax.rem(j, 2)
    nxt = 1 - slot

    @pl.when(j == 0)
    def _():
        _start_idx_dma(ei_ref, idx_sm, sems, 0, blk * be, be)

    @pl.when(j + 1 < nb)
    def _():
        _start_idx_dma(ei_ref, idx_sm, sems, nxt, (blk + 1) * be, be)

    _wait_idx_dma(ei_ref, idx_sm, sems, slot, blk * be, be)

    iota_h = jax.lax.broadcasted_iota(jnp.int32, (8, 16), 0)
    masks = [iota_h == u for u in range(8)]
    sbase = slot * be

    def gbody(k, _):
        k32 = pl.multiple_of(k * 32, 8)
        for cc in range(4):
            k8 = pl.multiple_of(k32 + cc * 8, 8)
            base = sbase + k8
            rows = []
            for u in range(8):
                pk = idx_sm[base + u]
                si = pk & 16383
                di = pk >> 14
                row = ap1_ref[si, 0, :] + ap2_ref[di, 0, :]
                rows.append(jnp.where(masks[u], row, 0.0))
            pre_scr[pl.ds(k8, 8), :] = _tree_sum(rows)
        return 0

    jax.lax.fori_loop(0, be // 32, gbody, 0)

    pre = (pre_scr[...]
           + jnp.dot(ea_ref[...], w7_ref[...],
                     preferred_element_type=jnp.float32)
           + b1_ref[...])
    h = pre * _sigmoid(pre)
    o = jnp.dot(h, w2_ref[...], preferred_element_type=jnp.float32) + b2_ref[...]
    out_ref[...] = o * _sigmoid(o)


def kernel(atom_fea, edge_index, edge_fea, distance,
           wf, bf, ws, bs, w1, b1, w2, b2):
    N, F = atom_fea.shape
    E, D = edge_fea.shape
    H = w1.shape[1]
    Dout = w2.shape[1]
    Hp = 16

    x = atom_fea.astype(jnp.float32)
    ea = edge_fea.astype(jnp.float32)
    dd = distance.astype(jnp.float32).reshape(E, 1)
    eidx = edge_index.astype(jnp.int32)
    ei = (eidx[1] << 14) | eidx[0]

    wf = wf.astype(jnp.float32)
    ws = ws.astype(jnp.float32)
    w1 = w1.astype(jnp.float32)
    w2 = w2.astype(jnp.float32)

    w4 = jnp.concatenate([wf[0:F], ws[0:F], wf[F:2 * F], ws[F:2 * F]], axis=1)
    b4 = jnp.concatenate([bf.astype(jnp.float32), bs.astype(jnp.float32),
                          jnp.zeros((2 * F,), jnp.float32)]).reshape(1, 4 * F)
    w5 = jnp.concatenate([wf[2 * F:], ws[2 * F:]], axis=1)

    w11p = jnp.pad(w1[0:F], ((0, 0), (0, Hp - H)))
    w12p = jnp.pad(w1[F:2 * F], ((0, 0), (0, Hp - H)))
    w6 = jnp.concatenate([w11p, w12p], axis=1)
    w7 = jnp.pad(w1[2 * F:], ((0, 0), (0, Hp - H)))
    b1p = jnp.pad(b1.astype(jnp.float32), (0, Hp - H)).reshape(1, Hp)
    w2p = jnp.pad(w2, ((0, Hp - H), (0, 0)))
    b2r = b2.astype(jnp.float32).reshape(1, Dout)

    bn = N // 2
    p_nodes = pl.pallas_call(
        _proj_kernel,
        out_shape=jax.ShapeDtypeStruct((N, 4 * F), jnp.float32),
        grid=(2,),
        in_specs=[
            pl.BlockSpec((bn, F), lambda i: (i, 0)),
            pl.BlockSpec((F, 4 * F), lambda i: (0, 0)),
            pl.BlockSpec((1, 4 * F), lambda i: (0, 0)),
        ],
        out_specs=pl.BlockSpec((bn, 4 * F), lambda i: (i, 0)),
        compiler_params=pltpu.CompilerParams(
            dimension_semantics=("arbitrary",)),
    )(x, w4, b4)
    p_nodes = p_nodes.reshape(N, 1, 4 * F)

    be = 2048 if E % 4096 == 0 else E // 2
    nb = E // (2 * be)
    parts = pl.pallas_call(
        functools.partial(_msg_kernel, be=be, nb=nb, f=F),
        out_shape=tuple(
            jax.ShapeDtypeStruct((2 * N, 1, 2 * F), jnp.float32)
            for _ in range(4)),
        grid=(2, nb + 1),
        in_specs=[
            pl.BlockSpec(memory_space=pl.ANY),
            pl.BlockSpec((be, D),
                         lambda c, j: (c * nb + jnp.minimum(j, nb - 1), 0)),
            pl.BlockSpec((be, 1),
                         lambda c, j: (c * nb + jnp.minimum(j, nb - 1), 0)),
            pl.BlockSpec((N, 1, 4 * F), lambda c, j: (0, 0, 0)),
            pl.BlockSpec((D, 2 * F), lambda c, j: (0, 0)),
        ],
        out_specs=tuple(
            pl.BlockSpec((N, 1, 2 * F), lambda c, j: (c, 0, 0))
            for _ in range(4)),
        scratch_shapes=[
            pltpu.VMEM((be, 2 * F), jnp.float32),
            pltpu.VMEM((2, be, F), jnp.float32),
            pltpu.SMEM((3 * be,), jnp.int32),
            pltpu.SemaphoreType.DMA((3,)),
        ],
        compiler_params=pltpu.CompilerParams(
            dimension_semantics=("arbitrary", "arbitrary")),
    )(ei, ea, dd, p_nodes, w5)
    parts2 = [p.reshape(2 * N, 2 * F) for p in parts]

    bn0 = N // 8
    atom_out, ap1, ap2 = pl.pallas_call(
        _fin_kernel,
        out_shape=(
            jax.ShapeDtypeStruct((N, F), jnp.float32),
            jax.ShapeDtypeStruct((N, Hp), jnp.float32),
            jax.ShapeDtypeStruct((N, Hp), jnp.float32),
        ),
        grid=(2, 4),
        in_specs=(
            [pl.BlockSpec((bn0, 2 * F), lambda c, i: (c * 4 + i, 0)),
             pl.BlockSpec((bn0, 2 * F), lambda c, i: (8 + c * 4 + i, 0))] * 4
            + [pl.BlockSpec((bn0, F), lambda c, i: (c * 4 + i, 0)),
               pl.BlockSpec((F, 2 * Hp), lambda c, i: (0, 0))]),
        out_specs=(
            pl.BlockSpec((bn0, F), lambda c, i: (c * 4 + i, 0)),
            pl.BlockSpec((bn0, Hp), lambda c, i: (c * 4 + i, 0)),
            pl.BlockSpec((bn0, Hp), lambda c, i: (c * 4 + i, 0)),
        ),
        compiler_params=pltpu.CompilerParams(
            dimension_semantics=("arbitrary", "arbitrary")),
    )(parts2[0], parts2[0], parts2[1], parts2[1],
      parts2[2], parts2[2], parts2[3], parts2[3], x, w6)
    ap1 = ap1.reshape(N, 1, Hp)
    ap2 = ap2.reshape(N, 1, Hp)

    edge_out = pl.pallas_call(
        functools.partial(_emlp_kernel, be=be, nb=nb),
        out_shape=jax.ShapeDtypeStruct((E, Dout), jnp.float32),
        grid=(2, nb),
        in_specs=[
            pl.BlockSpec(memory_space=pl.ANY),
            pl.BlockSpec((be, D), lambda c, j: (c * nb + j, 0)),
            pl.BlockSpec((N, 1, Hp), lambda c, j: (0, 0, 0)),
            pl.BlockSpec((N, 1, Hp), lambda c, j: (0, 0, 0)),
            pl.BlockSpec((D, Hp), lambda c, j: (0, 0)),
            pl.BlockSpec((1, Hp), lambda c, j: (0, 0)),
            pl.BlockSpec((Hp, Dout), lambda c, j: (0, 0)),
            pl.BlockSpec((1, Dout), lambda c, j: (0, 0)),
        ],
        out_specs=pl.BlockSpec((be, Dout), lambda c, j: (c * nb + j, 0)),
        scratch_shapes=[
            pltpu.VMEM((be, Hp), jnp.float32),
            pltpu.SMEM((2 * be,), jnp.int32),
            pltpu.SemaphoreType.DMA((2,)),
        ],
        compiler_params=pltpu.CompilerParams(
            dimension_semantics=("arbitrary", "arbitrary")),
    )(ei, ea, ap1, ap2, w7, b1p, w2p, b2r)

    return atom_out, edge_out


# speedup vs baseline: 4.0651x; 1.0093x over previous
import functools

import jax
import jax.numpy as jnp
from jax.experimental import pallas as pl
from jax.experimental.pallas import tpu as pltpu


def _sigmoid(x):
    return 1.0 / (1.0 + jnp.exp(-x))


def _softplus(x):
    return jnp.maximum(x, 0.0) + jnp.log(1.0 + jnp.exp(-jnp.abs(x)))


def _proj_kernel(x_ref, w_ref, b_ref, out_ref):
    out_ref[...] = (
        jnp.dot(x_ref[...], w_ref[...], preferred_element_type=jnp.float32)
        + b_ref[...])


def _start_idx_dma(ei_ref, idx_sm, sems, slot, start, be):
    pltpu.make_async_copy(
        ei_ref.at[pl.ds(pl.multiple_of(start, 1024), be)],
        idx_sm.at[pl.ds(slot * be, be)], sems.at[slot]).start()


def _wait_idx_dma(ei_ref, idx_sm, sems, slot, start, be):
    pltpu.make_async_copy(
        ei_ref.at[pl.ds(pl.multiple_of(start, 1024), be)],
        idx_sm.at[pl.ds(slot * be, be)], sems.at[slot]).wait()


def _tree_sum(vals):
    while len(vals) > 1:
        nxt = [a + b for a, b in zip(vals[0::2], vals[1::2])]
        if len(vals) % 2:
            nxt.append(vals[-1])
        vals = nxt
    return vals[0]


def _msg_kernel(ei_ref, ea_ref, d_ref, p_ref, w5_ref,
                outa_ref, outb_ref, outc_ref, outd_ref,
                z_scr, msg_scr, idx_sm, sems, *, be, nb, f):
    c = pl.program_id(0)
    j = pl.program_id(1)
    blk = c * nb + j
    slot = jax.lax.rem(j, 3)
    nxt = jax.lax.rem(j + 1, 3)
    prv = jax.lax.rem(j + 2, 3)

    @pl.when(j == 0)
    def _():
        _start_idx_dma(ei_ref, idx_sm, sems, 0, blk * be, be)
        outa_ref[...] = jnp.zeros_like(outa_ref)
        outb_ref[...] = jnp.zeros_like(outb_ref)
        outc_ref[...] = jnp.zeros_like(outc_ref)
        outd_ref[...] = jnp.zeros_like(outd_ref)

    @pl.when(j + 1 < nb)
    def _():
        _start_idx_dma(ei_ref, idx_sm, sems, nxt, (blk + 1) * be, be)

    @pl.when(j < nb)
    def _():
        _wait_idx_dma(ei_ref, idx_sm, sems, slot, blk * be, be)

    iota_z = jax.lax.broadcasted_iota(jnp.int32, (8, 2 * f), 0)
    masks = [iota_z == u for u in range(8)]
    sbase = slot * be
    pbase = prv * be
    mslot = jax.lax.rem(j, 2)
    mprv = 1 - mslot
    bufs = [outa_ref, outb_ref, outc_ref, outd_ref]

    def gather_chunk(k8, base):
        rows = []
        for u in range(8):
            pk = idx_sm[base + u]
            si = pk & 16383
            di = pk >> 14
            row = p_ref[di, 0, 0:2 * f] + p_ref[si, 0, 2 * f:4 * f]
            rows.append(jnp.where(masks[u], row, 0.0))
        z_scr[pl.ds(k8, 8), :] = _tree_sum(rows)

    def scatter_chunk(k8, base):
        chm = msg_scr[mprv, pl.ds(k8, 8), :]
        for u in range(8):
            di = idx_sm[base + u] >> 14
            row = chm[u]
            tgt = bufs[u % 4]
            tgt[di, 0, 0:f] = tgt[di, 0, 0:f] + row

    @pl.when((j > 0) & (j < nb))
    def _():
        def fbody(k, _):
            k32 = pl.multiple_of(k * 32, 8)
            for cc in range(4):
                k8 = pl.multiple_of(k32 + cc * 8, 8)
                gather_chunk(k8, sbase + k8)
                scatter_chunk(k8, pbase + k8)
            return 0
        jax.lax.fori_loop(0, be // 32, fbody, 0)

    @pl.when(j == 0)
    def _():
        def gb(k, _):
            k16 = pl.multiple_of(k * 16, 8)
            for cc in range(2):
                k8 = pl.multiple_of(k16 + cc * 8, 8)
                gather_chunk(k8, sbase + k8)
            return 0
        jax.lax.fori_loop(0, be // 16, gb, 0)

    @pl.when(j == nb)
    def _():
        def sb(k, _):
            k16 = pl.multiple_of(k * 16, 8)
            for cc in range(2):
                k8 = pl.multiple_of(k16 + cc * 8, 8)
                scatter_chunk(k8, pbase + k8)
            return 0
        jax.lax.fori_loop(0, be // 16, sb, 0)

    @pl.when(j < nb)
    def _():
        q = jnp.dot(ea_ref[...], w5_ref[...],
                    preferred_element_type=jnp.float32)
        z = z_scr[...] + q
        zf = z[:, 0:f]
        zs = z[:, f:2 * f]
        d = d_ref[...]
        g = jnp.exp(d * d * (-1.0 / 18.0))
        msg_scr[mslot, :, :] = _sigmoid(zf) * _softplus(zs) * g


def _fin_kernel(pa0_ref, pa1_ref, pb0_ref, pb1_ref,
                pc0_ref, pc1_ref, pd0_ref, pd1_ref, x_ref, w6_ref,
                atom_ref, ap1_ref, ap2_ref):
    fdim = x_ref.shape[1]
    psum = (((pa0_ref[...] + pa1_ref[...]) + (pb0_ref[...] + pb1_ref[...]))
            + ((pc0_ref[...] + pc1_ref[...]) + (pd0_ref[...] + pd1_ref[...])))
    a = psum[:, 0:fdim] + x_ref[...]
    atom_ref[...] = a
    ap = jnp.dot(a, w6_ref[...], preferred_element_type=jnp.float32)
    ap1_ref[...] = ap[:, 0:16]
    ap2_ref[...] = ap[:, 16:32]


def _emlp_kernel(ei_ref, ea_ref, ap1_ref, ap2_ref, w7_ref, b1_ref,
                 w2_ref, b2_ref, out_ref, pre_scr, idx_sm, sems, *, be, nb):
    c = pl.program_id(0)
    j = pl.program_id(1)
    blk = c * nb + j
    slot = jax.lax.rem(j, 2)
    nxt = 1 - slot

    @pl.when(j == 0)
    def _():
        _start_idx_dma(ei_ref, idx_sm, sems, 0, blk * be, be)

    @pl.when(j + 1 < nb)
    def _():
        _start_idx_dma(ei_ref, idx_sm, sems, nxt, (blk + 1) * be, be)

    _wait_idx_dma(ei_ref, idx_sm, sems, slot, blk * be, be)

    iota_h = jax.lax.broadcasted_iota(jnp.int32, (8, 16), 0)
    masks = [iota_h == u for u in range(8)]
    sbase = slot * be

    def gbody(k, _):
        k32 = pl.multiple_of(k * 32, 8)
        for cc in range(4):
            k8 = pl.multiple_of(k32 + cc * 8, 8)
            base = sbase + k8
            rows = []
            for u in range(8):
                pk = idx_sm[base + u]
                si = pk & 16383
                di = pk >> 14
                row = ap1_ref[si, 0, :] + ap2_ref[di, 0, :]
                rows.append(jnp.where(masks[u], row, 0.0))
            pre_scr[pl.ds(k8, 8), :] = _tree_sum(rows)
        return 0

    jax.lax.fori_loop(0, be // 32, gbody, 0)

    pre = (pre_scr[...]
           + jnp.dot(ea_ref[...], w7_ref[...],
                     preferred_element_type=jnp.float32)
           + b1_ref[...])
    h = pre * _sigmoid(pre)
    o = jnp.dot(h, w2_ref[...], preferred_element_type=jnp.float32) + b2_ref[...]
    out_ref[...] = o * _sigmoid(o)


def kernel(atom_fea, edge_index, edge_fea, distance,
           wf, bf, ws, bs, w1, b1, w2, b2):
    N, F = atom_fea.shape
    E, D = edge_fea.shape
    H = w1.shape[1]
    Dout = w2.shape[1]
    Hp = 16

    x = atom_fea.astype(jnp.float32)
    ea = edge_fea.astype(jnp.float32)
    dd = distance.astype(jnp.float32).reshape(E, 1)
    eidx = edge_index.astype(jnp.int32)
    ei = (eidx[1] << 14) | eidx[0]

    wf = wf.astype(jnp.float32)
    ws = ws.astype(jnp.float32)
    w1 = w1.astype(jnp.float32)
    w2 = w2.astype(jnp.float32)

    w4 = jnp.concatenate([wf[0:F], ws[0:F], wf[F:2 * F], ws[F:2 * F]], axis=1)
    b4 = jnp.concatenate([bf.astype(jnp.float32), bs.astype(jnp.float32),
                          jnp.zeros((2 * F,), jnp.float32)]).reshape(1, 4 * F)
    w5 = jnp.concatenate([wf[2 * F:], ws[2 * F:]], axis=1)

    w11p = jnp.pad(w1[0:F], ((0, 0), (0, Hp - H)))
    w12p = jnp.pad(w1[F:2 * F], ((0, 0), (0, Hp - H)))
    w6 = jnp.concatenate([w11p, w12p], axis=1)
    w7 = jnp.pad(w1[2 * F:], ((0, 0), (0, Hp - H)))
    b1p = jnp.pad(b1.astype(jnp.float32), (0, Hp - H)).reshape(1, Hp)
    w2p = jnp.pad(w2, ((0, Hp - H), (0, 0)))
    b2r = b2.astype(jnp.float32).reshape(1, Dout)

    bn = N // 2
    p_nodes = pl.pallas_call(
        _proj_kernel,
        out_shape=jax.ShapeDtypeStruct((N, 4 * F), jnp.float32),
        grid=(2,),
        in_specs=[
            pl.BlockSpec((bn, F), lambda i: (i, 0)),
            pl.BlockSpec((F, 4 * F), lambda i: (0, 0)),
            pl.BlockSpec((1, 4 * F), lambda i: (0, 0)),
        ],
        out_specs=pl.BlockSpec((bn, 4 * F), lambda i: (i, 0)),
        compiler_params=pltpu.CompilerParams(
            dimension_semantics=("arbitrary",)),
    )(x, w4, b4)
    p_nodes = p_nodes.reshape(N, 1, 4 * F)

    if E % 8192 == 0:
        be = 4096
    elif E % 4096 == 0:
        be = 2048
    else:
        be = E // 2
    nb = E // (2 * be)
    parts = pl.pallas_call(
        functools.partial(_msg_kernel, be=be, nb=nb, f=F),
        out_shape=tuple(
            jax.ShapeDtypeStruct((2 * N, 1, 2 * F), jnp.float32)
            for _ in range(4)),
        grid=(2, nb + 1),
        in_specs=[
            pl.BlockSpec(memory_space=pl.ANY),
            pl.BlockSpec((be, D),
                         lambda c, j: (c * nb + jnp.minimum(j, nb - 1), 0)),
            pl.BlockSpec((be, 1),
                         lambda c, j: (c * nb + jnp.minimum(j, nb - 1), 0)),
            pl.BlockSpec((N, 1, 4 * F), lambda c, j: (0, 0, 0)),
            pl.BlockSpec((D, 2 * F), lambda c, j: (0, 0)),
        ],
        out_specs=tuple(
            pl.BlockSpec((N, 1, 2 * F), lambda c, j: (c, 0, 0))
            for _ in range(4)),
        scratch_shapes=[
            pltpu.VMEM((be, 2 * F), jnp.float32),
            pltpu.VMEM((2, be, F), jnp.float32),
            pltpu.SMEM((3 * be,), jnp.int32),
            pltpu.SemaphoreType.DMA((3,)),
        ],
        compiler_params=pltpu.CompilerParams(
            dimension_semantics=("arbitrary", "arbitrary")),
    )(ei, ea, dd, p_nodes, w5)
    parts2 = [p.reshape(2 * N, 2 * F) for p in parts]

    bn0 = N // 8
    atom_out, ap1, ap2 = pl.pallas_call(
        _fin_kernel,
        out_shape=(
            jax.ShapeDtypeStruct((N, F), jnp.float32),
            jax.ShapeDtypeStruct((N, Hp), jnp.float32),
            jax.ShapeDtypeStruct((N, Hp), jnp.float32),
        ),
        grid=(2, 4),
        in_specs=(
            [pl.BlockSpec((bn0, 2 * F), lambda c, i: (c * 4 + i, 0)),
             pl.BlockSpec((bn0, 2 * F), lambda c, i: (8 + c * 4 + i, 0))] * 4
            + [pl.BlockSpec((bn0, F), lambda c, i: (c * 4 + i, 0)),
               pl.BlockSpec((F, 2 * Hp), lambda c, i: (0, 0))]),
        out_specs=(
            pl.BlockSpec((bn0, F), lambda c, i: (c * 4 + i, 0)),
            pl.BlockSpec((bn0, Hp), lambda c, i: (c * 4 + i, 0)),
            pl.BlockSpec((bn0, Hp), lambda c, i: (c * 4 + i, 0)),
        ),
        compiler_params=pltpu.CompilerParams(
            dimension_semantics=("arbitrary", "arbitrary")),
    )(parts2[0], parts2[0], parts2[1], parts2[1],
      parts2[2], parts2[2], parts2[3], parts2[3], x, w6)
    ap1 = ap1.reshape(N, 1, Hp)
    ap2 = ap2.reshape(N, 1, Hp)

    edge_out = pl.pallas_call(
        functools.partial(_emlp_kernel, be=be, nb=nb),
        out_shape=jax.ShapeDtypeStruct((E, Dout), jnp.float32),
        grid=(2, nb),
        in_specs=[
            pl.BlockSpec(memory_space=pl.ANY),
            pl.BlockSpec((be, D), lambda c, j: (c * nb + j, 0)),
            pl.BlockSpec((N, 1, Hp), lambda c, j: (0, 0, 0)),
            pl.BlockSpec((N, 1, Hp), lambda c, j: (0, 0, 0)),
            pl.BlockSpec((D, Hp), lambda c, j: (0, 0)),
            pl.BlockSpec((1, Hp), lambda c, j: (0, 0)),
            pl.BlockSpec((Hp, Dout), lambda c, j: (0, 0)),
            pl.BlockSpec((1, Dout), lambda c, j: (0, 0)),
        ],
        out_specs=pl.BlockSpec((be, Dout), lambda c, j: (c * nb + j, 0)),
        scratch_shapes=[
            pltpu.VMEM((be, Hp), jnp.float32),
            pltpu.SMEM((2 * be,), jnp.int32),
            pltpu.SemaphoreType.DMA((2,)),
        ],
        compiler_params=pltpu.CompilerParams(
            dimension_semantics=("arbitrary", "arbitrary")),
    )(ei, ea, ap1, ap2, w7, b1p, w2p, b2r)

    return atom_out, edge_out
